# baseline (speedup 1.0000x reference)
import jax
import jax.numpy as jnp
from jax.experimental import pallas as pl

BS = 64
SEQ_LEN = 2048
PRED_LEN = 1024
C_IN = 32
SEG = 16
NUM_MAP = 16
KSIZE = 16
STRIDE = 8
CONV_DIM = (SEQ_LEN - KSIZE) // STRIDE + 1
N_IN = SEQ_LEN // SEG
N_OUT = PRED_LEN // SEG
NB = 16
LAN = NB * C_IN
SC = SEG * C_IN


def _step(x_ref, x3_ref, cw_ref, cb_ref, gw_ref, gb_ref, W_ref, out_ref):
    xb = jnp.concatenate([x_ref[nb] for nb in range(NB)], axis=1)
    mean = jnp.mean(xb, axis=0, keepdims=True)
    xc = xb - mean
    var = jnp.mean(xc * xc, axis=0, keepdims=True)
    std = jnp.sqrt(var + 1e-10)
    xn = xc / std

    r = xn.astype(jnp.bfloat16).astype(jnp.float32).reshape(256, 8, LAN)
    acc = jnp.zeros((CONV_DIM, LAN), jnp.float32)
    for k in range(KSIZE):
        sl = r[0:255, k, :] if k < 8 else r[1:256, k - 8, :]
        acc = acc + sl * cw_ref[k:k + 1, :]
    conv = acc + cb_ref[...]

    logits = jnp.dot(gw_ref[...].astype(jnp.bfloat16),
                     conv.astype(jnp.bfloat16),
                     preferred_element_type=jnp.float32)
    logits = logits + gb_ref[...]
    mx = jnp.max(logits, axis=0, keepdims=True)
    e = jnp.exp(logits - mx)
    g = e / jnp.sum(e, axis=0, keepdims=True)

    miota = jax.lax.broadcasted_iota(jnp.int32, (NUM_MAP, LAN), 0)
    v1 = jnp.max(g, axis=0, keepdims=True)
    i1 = jnp.min(jnp.where(g == v1, miota, NUM_MAP), axis=0, keepdims=True)
    mask1 = miota == i1
    g2 = jnp.where(mask1, -1.0, g)
    v2 = jnp.max(g2, axis=0, keepdims=True)
    i2 = jnp.min(jnp.where(g2 == v2, miota, NUM_MAP), axis=0, keepdims=True)
    mask2 = miota == i2
    p1 = 1.0 / (1.0 + jnp.exp(v2 - v1))
    gf = jnp.where(mask1, p1, 0.0) + jnp.where(mask2, 1.0 - p1, 0.0)

    si = jax.lax.broadcasted_iota(jnp.int32, (NUM_MAP * N_OUT, NUM_MAP), 0)
    sj = jax.lax.broadcasted_iota(jnp.int32, (NUM_MAP * N_OUT, NUM_MAP), 1)
    bsel = (si // N_OUT == sj).astype(jnp.float32)
    ones = jnp.ones((1, SC), jnp.float32)

    for nb in range(NB):
        mrow = jnp.tile(mean[:, nb * C_IN:(nb + 1) * C_IN], (1, SEG))
        srow = jnp.tile(std[:, nb * C_IN:(nb + 1) * C_IN], (1, SEG))
        gft = jnp.tile(gf[:, nb * C_IN:(nb + 1) * C_IN], (1, SEG))
        xs = (x3_ref[nb] - mrow) / srow
        xsa = jnp.concatenate([xs, ones], axis=0)
        y = jnp.dot(W_ref[...], xsa, preferred_element_type=jnp.float32)
        gbig = jnp.dot(bsel, gft, preferred_element_type=jnp.float32)
        z = y * gbig
        f = z[0:512, :] + z[512:1024, :]
        f = f[0:256, :] + f[256:512, :]
        f = f[0:128, :] + f[128:256, :]
        o = f[0:N_OUT, :] + f[N_OUT:2 * N_OUT, :]
        out_ref[nb] = o * srow + mrow


def kernel(x, conv_w, conv_b, gate_w, gate_b, map_w, map_b):
    x3 = x.reshape(BS, N_IN, SC)
    cw = conv_w[:, 0, :].T
    cwt = jnp.tile(cw, (1, NB))
    cbt = jnp.tile(conv_b, (NB,)).reshape(1, LAN)
    gb = gate_b.reshape(NUM_MAP, 1)
    waug = jnp.concatenate(
        [map_w.reshape(NUM_MAP * N_OUT, N_IN), map_b.reshape(NUM_MAP * N_OUT, 1)],
        axis=1)

    res = pl.pallas_call(
        _step,
        grid=(BS // NB,),
        in_specs=[
            pl.BlockSpec((NB, SEQ_LEN, C_IN), lambda i: (i, 0, 0)),
            pl.BlockSpec((NB, N_IN, SC), lambda i: (i, 0, 0)),
            pl.BlockSpec((KSIZE, LAN), lambda i: (0, 0)),
            pl.BlockSpec((1, LAN), lambda i: (0, 0)),
            pl.BlockSpec((NUM_MAP, CONV_DIM), lambda i: (0, 0)),
            pl.BlockSpec((NUM_MAP, 1), lambda i: (0, 0)),
            pl.BlockSpec((NUM_MAP * N_OUT, N_IN + 1), lambda i: (0, 0)),
        ],
        out_specs=pl.BlockSpec((NB, N_OUT, SC), lambda i: (i, 0, 0)),
        out_shape=jax.ShapeDtypeStruct((BS, N_OUT, SC), jnp.float32),
    )(x, x3, cwt, cbt, gate_w, gb, waug)

    return res.reshape(BS, PRED_LEN, C_IN)


# speedup vs baseline: 1.0158x; 1.0158x over previous
import jax
import jax.numpy as jnp
from jax.experimental import pallas as pl

BS = 64
SEQ_LEN = 2048
PRED_LEN = 1024
C_IN = 32
SEG = 16
NUM_MAP = 16
KSIZE = 16
STRIDE = 8
CONV_DIM = (SEQ_LEN - KSIZE) // STRIDE + 1
N_IN = SEQ_LEN // SEG
N_OUT = PRED_LEN // SEG
NB = 8
LAN = NB * C_IN
SC = SEG * C_IN


def _step(x_ref, x3_ref, cw_ref, cb_ref, gw_ref, gb_ref, W_ref, out_ref):
    xb = jnp.concatenate([x_ref[nb] for nb in range(NB)], axis=1)
    mean = jnp.mean(xb, axis=0, keepdims=True)
    xc = xb - mean
    var = jnp.mean(xc * xc, axis=0, keepdims=True)
    std = jnp.sqrt(var + 1e-10)
    xn = xc / std

    r = xn.astype(jnp.bfloat16).astype(jnp.float32).reshape(256, 8, LAN)
    acc = jnp.zeros((CONV_DIM, LAN), jnp.float32)
    for k in range(KSIZE):
        sl = r[0:255, k, :] if k < 8 else r[1:256, k - 8, :]
        acc = acc + sl * cw_ref[k:k + 1, :]
    conv = acc + cb_ref[...]

    logits = jnp.dot(gw_ref[...].astype(jnp.bfloat16),
                     conv.astype(jnp.bfloat16),
                     preferred_element_type=jnp.float32)
    logits = logits + gb_ref[...]
    mx = jnp.max(logits, axis=0, keepdims=True)
    e = jnp.exp(logits - mx)
    g = e / jnp.sum(e, axis=0, keepdims=True)

    miota = jax.lax.broadcasted_iota(jnp.int32, (NUM_MAP, LAN), 0)
    v1 = jnp.max(g, axis=0, keepdims=True)
    i1 = jnp.min(jnp.where(g == v1, miota, NUM_MAP), axis=0, keepdims=True)
    mask1 = miota == i1
    g2 = jnp.where(mask1, -1.0, g)
    v2 = jnp.max(g2, axis=0, keepdims=True)
    i2 = jnp.min(jnp.where(g2 == v2, miota, NUM_MAP), axis=0, keepdims=True)
    mask2 = miota == i2
    p1 = 1.0 / (1.0 + jnp.exp(v2 - v1))
    gf = jnp.where(mask1, p1, 0.0) + jnp.where(mask2, 1.0 - p1, 0.0)

    si = jax.lax.broadcasted_iota(jnp.int32, (NUM_MAP * N_OUT, NUM_MAP), 0)
    sj = jax.lax.broadcasted_iota(jnp.int32, (NUM_MAP * N_OUT, NUM_MAP), 1)
    bsel = (si // N_OUT == sj).astype(jnp.float32)
    ones = jnp.ones((1, SC), jnp.float32)

    for nb in range(NB):
        mrow = jnp.tile(mean[:, nb * C_IN:(nb + 1) * C_IN], (1, SEG))
        srow = jnp.tile(std[:, nb * C_IN:(nb + 1) * C_IN], (1, SEG))
        gft = jnp.tile(gf[:, nb * C_IN:(nb + 1) * C_IN], (1, SEG))
        xs = (x3_ref[nb] - mrow) / srow
        xsa = jnp.concatenate([xs, ones], axis=0)
        y = jnp.dot(W_ref[...], xsa, preferred_element_type=jnp.float32)
        gbig = jnp.dot(bsel, gft, preferred_element_type=jnp.float32)
        z = y * gbig
        f = z[0:512, :] + z[512:1024, :]
        f = f[0:256, :] + f[256:512, :]
        f = f[0:128, :] + f[128:256, :]
        o = f[0:N_OUT, :] + f[N_OUT:2 * N_OUT, :]
        out_ref[nb] = o * srow + mrow


def kernel(x, conv_w, conv_b, gate_w, gate_b, map_w, map_b):
    x3 = x.reshape(BS, N_IN, SC)
    cw = conv_w[:, 0, :].T
    cwt = jnp.tile(cw, (1, NB))
    cbt = jnp.tile(conv_b, (NB,)).reshape(1, LAN)
    gb = gate_b.reshape(NUM_MAP, 1)
    waug = jnp.concatenate(
        [map_w.reshape(NUM_MAP * N_OUT, N_IN), map_b.reshape(NUM_MAP * N_OUT, 1)],
        axis=1)

    res = pl.pallas_call(
        _step,
        grid=(BS // NB,),
        in_specs=[
            pl.BlockSpec((NB, SEQ_LEN, C_IN), lambda i: (i, 0, 0)),
            pl.BlockSpec((NB, N_IN, SC), lambda i: (i, 0, 0)),
            pl.BlockSpec((KSIZE, LAN), lambda i: (0, 0)),
            pl.BlockSpec((1, LAN), lambda i: (0, 0)),
            pl.BlockSpec((NUM_MAP, CONV_DIM), lambda i: (0, 0)),
            pl.BlockSpec((NUM_MAP, 1), lambda i: (0, 0)),
            pl.BlockSpec((NUM_MAP * N_OUT, N_IN + 1), lambda i: (0, 0)),
        ],
        out_specs=pl.BlockSpec((NB, N_OUT, SC), lambda i: (i, 0, 0)),
        out_shape=jax.ShapeDtypeStruct((BS, N_OUT, SC), jnp.float32),
    )(x, x3, cwt, cbt, gate_w, gb, waug)

    return res.reshape(BS, PRED_LEN, C_IN)
